# baseline (speedup 1.0000x reference)
import jax
import jax.numpy as jnp
import numpy as np
from jax import lax
from jax.experimental import pallas as pl
from jax.experimental.pallas import tpu as pltpu

_E = 4
_D = 16
_GRID = 8
_DEG = 3
_C = 16384

_H = 1.2 / _GRID
_NKNOT = _GRID + 2 * _DEG + 1
_KLO = np.float32(-0.1 - _DEG * _H)
_KHI = np.float32(1.1 + _DEG * _H)
_KSTEP = np.float32((_KHI - _KLO) / (_NKNOT - 1))
_EPS = 1e-8
_NB = _GRID + _DEG
_NFP = 80
_OFF_SPL, _OFF_SILU, _OFF_COS, _OFF_SIN, _OFF_G, _OFF_W = 0, 11, 16, 32, 48, 64

_CPOLY = [np.float32(c) for c in (
    1.0, -19.739204, 64.93912, -85.45011, 60.16743, -25.966885, 6.527706)]

_DNT = (((0,), (0,)), ((), ()))
_DNR = (((1,), (0,)), ((), ()))


def _cos2pi(w):
    f = w - jnp.floor(w + 0.5)
    y = f * f
    acc = _CPOLY[6]
    for c in (_CPOLY[5], _CPOLY[4], _CPOLY[3], _CPOLY[2], _CPOLY[1], _CPOLY[0]):
        acc = acc * y + c
    return acc


def _kmote_block(t_ref, aux_ref, w1_ref, b1_ref, w2t_ref, b2_ref,
                 af_ref, bf_ref, wbs_ref, ws_ref, cg_ref, sg_ref,
                 sw_ref, cw_ref, wg_ref, ww_ref,
                 emb_ref, wts_ref, mask_ref, wbig_ref):
    @pl.when(pl.program_id(0) == 0)
    def _init():
        wbig_ref[:, :] = jnp.zeros((_NFP, _E * _D), jnp.float32)
        wbig_ref[_OFF_SPL:_OFF_SPL + _NB, _D:2 * _D] = ws_ref[:, :]
        wbig_ref[_OFF_SILU:_OFF_SILU + 1, _D:2 * _D] = wbs_ref[:, :]
        wbig_ref[_OFF_COS:_OFF_COS + _D, 0:_D] = af_ref[:, :]
        wbig_ref[_OFF_SIN:_OFF_SIN + _D, 0:_D] = bf_ref[:, :]
        wbig_ref[_OFF_G:_OFF_G + _D, 2 * _D:3 * _D] = wg_ref[:, :]
        wbig_ref[_OFF_W:_OFF_W + _D, 3 * _D:4 * _D] = ww_ref[:, :]

    t = t_ref[:, :]
    aux_t = aux_ref[:, :]

    rin = jnp.concatenate([t, aux_t], axis=0)
    h = lax.dot_general(w1_ref[:, :], rin, _DNT,
                        preferred_element_type=jnp.float32) + b1_ref[:, :].T
    h = jnp.maximum(h, 0.0)
    logits = lax.dot_general(w2t_ref[:, :], h, _DNR,
                             preferred_element_type=jnp.float32) + b2_ref[:, :].T

    m = jnp.max(logits, axis=0, keepdims=True)
    ex = jnp.exp(logits - m)
    w = ex / jnp.sum(ex, axis=0, keepdims=True)

    iota = jax.lax.broadcasted_iota(jnp.int32, w.shape, 0)
    m1 = jnp.max(w, axis=0, keepdims=True)
    i1 = jnp.min(jnp.where(w == m1, iota, _E), axis=0, keepdims=True)
    w2nd = jnp.where(iota == i1, -1.0, w)
    m2 = jnp.max(w2nd, axis=0, keepdims=True)
    i2 = jnp.min(jnp.where(w2nd == m2, iota, _E), axis=0, keepdims=True)
    sel = (iota == i1) | (iota == i2)
    disp = jnp.where(sel, w, 0.0)
    d_f = disp[0:1, :]
    d_s = disp[1:2, :]
    d_g = disp[2:3, :]
    d_w = disp[3:4, :]

    kfreq = jax.lax.broadcasted_iota(jnp.int32, (_D, 1), 0).astype(jnp.float32) + 1.0
    kt = kfreq * t
    cosf = _cos2pi(kt) * d_f
    sinf = _cos2pi(kt - 0.25) * d_f

    dg = (t - cg_ref[:, :].T) / sg_ref[:, :].T
    phi = jnp.exp(-0.5 * dg * dg) * d_g

    u = (t - cw_ref[:, :].T) / sw_ref[:, :].T
    psi = _cos2pi(u * np.float32(5.0 / (2.0 * np.pi))) * jnp.exp(-0.5 * u * u) * d_w

    kn = _KLO + _KSTEP * jax.lax.broadcasted_iota(jnp.int32, (_NKNOT, 1), 0).astype(jnp.float32)
    b = ((t >= kn[:-1, :]) & (t < kn[1:, :])).astype(jnp.float32)
    for k in range(1, _DEG + 1):
        ldenom = 1.0 / (kn[k:-1, :] - kn[:-(k + 1), :] + _EPS)
        rdenom = 1.0 / (kn[(k + 1):, :] - kn[1:-k, :] + _EPS)
        left = (t - kn[:-(k + 1), :]) * ldenom * b[:-1, :]
        right = (kn[(k + 1):, :] - t) * rdenom * b[1:, :]
        b = left + right
    basis = b * d_s

    silu_t = t * (1.0 / (1.0 + jnp.exp(-t))) * d_s

    pad = jnp.zeros((_OFF_COS - _OFF_SILU - 1, t.shape[1]), dtype=jnp.float32)
    feats = jnp.concatenate([basis, silu_t, pad, cosf, sinf, phi, psi], axis=0)

    emb_ref[:, :] = lax.dot_general(wbig_ref[:, :], feats, _DNT,
                                    preferred_element_type=jnp.float32)
    wts_ref[:, :] = w
    mask_ref[:, :] = sel


def kernel(timestamp_input, auxiliary_features, W1, b1, W2, b2, A_f, B_f,
           Wb_s, W_s, C_g, Sig_g, W_g, S_w, C_w, W_w):
    Bsz = timestamp_input.shape[0]
    t_row = timestamp_input.reshape(1, Bsz)
    aux_t = auxiliary_features.T
    row = lambda v: v.reshape(1, -1)

    grid = (Bsz // _C,)
    full = lambda a: pl.BlockSpec(a.shape, lambda i: (0,) * a.ndim)

    emb_t, wts_t, mask_t = pl.pallas_call(
        _kmote_block,
        grid=grid,
        in_specs=[
            pl.BlockSpec((1, _C), lambda i: (0, i)),
            pl.BlockSpec((_D, _C), lambda i: (0, i)),
            full(W1), full(row(b1)), full(W2.T), full(row(b2)),
            full(A_f), full(B_f), full(Wb_s), full(W_s),
            full(row(C_g)), full(row(Sig_g)), full(row(S_w)), full(row(C_w)),
            full(W_g), full(W_w),
        ],
        out_specs=[
            pl.BlockSpec((_E * _D, _C), lambda i: (0, i)),
            pl.BlockSpec((_E, _C), lambda i: (0, i)),
            pl.BlockSpec((_E, _C), lambda i: (0, i)),
        ],
        out_shape=[
            jax.ShapeDtypeStruct((_E * _D, Bsz), jnp.float32),
            jax.ShapeDtypeStruct((_E, Bsz), jnp.float32),
            jax.ShapeDtypeStruct((_E, Bsz), jnp.bool_),
        ],
        scratch_shapes=[pltpu.VMEM((_NFP, _E * _D), jnp.float32)],
    )(t_row, aux_t, W1, row(b1), W2.T, row(b2),
      A_f, B_f, Wb_s, W_s, row(C_g), row(Sig_g), row(S_w), row(C_w),
      W_g, W_w)

    return (emb_t.T, wts_t.T, mask_t.T)


# speedup vs baseline: 1.1287x; 1.1287x over previous
import jax
import jax.numpy as jnp
import numpy as np
from jax import lax
from jax.experimental import pallas as pl
from jax.experimental.pallas import tpu as pltpu

_E = 4
_D = 16
_GRID = 8
_DEG = 3
_C = 8192

_H = 1.2 / _GRID
_NKNOT = _GRID + 2 * _DEG + 1
_KLO = np.float32(-0.1 - _DEG * _H)
_KHI = np.float32(1.1 + _DEG * _H)
_KSTEP = np.float32((_KHI - _KLO) / (_NKNOT - 1))
_EPS = 1e-8
_NB = _GRID + _DEG
_NFP = 80
_OFF_SPL, _OFF_SILU, _OFF_COS, _OFF_SIN, _OFF_G, _OFF_W = 0, 11, 16, 32, 48, 64

_CPOLY = [np.float32(c) for c in (
    0.9999582, -19.730843, 64.66944, -82.37804, 45.595547)]

_DNT = (((0,), (0,)), ((), ()))
_DNR = (((1,), (0,)), ((), ()))


def _cos2pi(w):
    f = w - jnp.floor(w + 0.5)
    y = f * f
    acc = _CPOLY[4]
    for c in (_CPOLY[3], _CPOLY[2], _CPOLY[1], _CPOLY[0]):
        acc = acc * y + c
    return acc


def _kmote_block(t_ref, aux_ref, w1_ref, b1_ref, w2t_ref, b2_ref,
                 af_ref, bf_ref, wbs_ref, ws_ref, cg_ref, sg_ref,
                 sw_ref, cw_ref, wg_ref, ww_ref,
                 emb_ref, wts_ref, mask_ref, wbig_ref):
    @pl.when(pl.program_id(0) == 0)
    def _init():
        wbig_ref[:, :] = jnp.zeros((_NFP, _E * _D), jnp.float32)
        wbig_ref[_OFF_SPL:_OFF_SPL + _NB, _D:2 * _D] = ws_ref[:, :]
        wbig_ref[_OFF_SILU:_OFF_SILU + 1, _D:2 * _D] = wbs_ref[:, :]
        wbig_ref[_OFF_COS:_OFF_COS + _D, 0:_D] = af_ref[:, :]
        wbig_ref[_OFF_SIN:_OFF_SIN + _D, 0:_D] = bf_ref[:, :]
        wbig_ref[_OFF_G:_OFF_G + _D, 2 * _D:3 * _D] = wg_ref[:, :]
        wbig_ref[_OFF_W:_OFF_W + _D, 3 * _D:4 * _D] = ww_ref[:, :]

    t = t_ref[:, :]
    aux_t = aux_ref[:, :]

    rin = jnp.concatenate([t, aux_t], axis=0)
    h = lax.dot_general(w1_ref[:, :], rin, _DNT,
                        preferred_element_type=jnp.float32) + b1_ref[:, :].T
    h = jnp.maximum(h, 0.0)
    logits = lax.dot_general(w2t_ref[:, :], h, _DNR,
                             preferred_element_type=jnp.float32) + b2_ref[:, :].T

    m = jnp.max(logits, axis=0, keepdims=True)
    ex = jnp.exp(logits - m)
    w = ex / jnp.sum(ex, axis=0, keepdims=True)

    iota = jax.lax.broadcasted_iota(jnp.int32, w.shape, 0)
    m1 = jnp.max(w, axis=0, keepdims=True)
    i1 = jnp.min(jnp.where(w == m1, iota, _E), axis=0, keepdims=True)
    w2nd = jnp.where(iota == i1, -1.0, w)
    m2 = jnp.max(w2nd, axis=0, keepdims=True)
    i2 = jnp.min(jnp.where(w2nd == m2, iota, _E), axis=0, keepdims=True)
    sel = (iota == i1) | (iota == i2)
    disp = jnp.where(sel, w, 0.0)
    d_f = disp[0:1, :]
    d_s = disp[1:2, :]
    d_g = disp[2:3, :]
    d_w = disp[3:4, :]

    kfreq = jax.lax.broadcasted_iota(jnp.int32, (_D, 1), 0).astype(jnp.float32) + 1.0
    kt = kfreq * t
    cosf = _cos2pi(kt) * d_f
    sinf = _cos2pi(kt - 0.25) * d_f

    dg = (t - cg_ref[:, :].T) / sg_ref[:, :].T
    phi = jnp.exp(-0.5 * dg * dg) * d_g

    u = (t - cw_ref[:, :].T) / sw_ref[:, :].T
    psi = _cos2pi(u * np.float32(5.0 / (2.0 * np.pi))) * jnp.exp(-0.5 * u * u) * d_w

    kvec = _KLO + _KSTEP * jax.lax.broadcasted_iota(jnp.int32, (_NB, 1), 0).astype(jnp.float32)
    wspl = (t - kvec) * np.float32(1.0 / _H)
    v = jnp.abs(wspl - 2.0)
    v2 = v * v
    edge = 2.0 - v
    inner = 4.0 + v2 * (3.0 * v - 6.0)
    b3 = jnp.where(v < 1.0, inner, jnp.where(v < 2.0, edge * edge * edge, 0.0))
    basis = b3 * (d_s * np.float32(1.0 / 6.0))

    silu_t = t * (1.0 / (1.0 + jnp.exp(-t))) * d_s

    pad = jnp.zeros((_OFF_COS - _OFF_SILU - 1, t.shape[1]), dtype=jnp.float32)
    feats = jnp.concatenate([basis, silu_t, pad, cosf, sinf, phi, psi], axis=0)

    emb_ref[:, :] = lax.dot_general(wbig_ref[:, :], feats, _DNT,
                                    preferred_element_type=jnp.float32)
    wts_ref[:, :] = w
    mask_ref[:, :] = sel


def kernel(timestamp_input, auxiliary_features, W1, b1, W2, b2, A_f, B_f,
           Wb_s, W_s, C_g, Sig_g, W_g, S_w, C_w, W_w):
    Bsz = timestamp_input.shape[0]
    t_row = timestamp_input.reshape(1, Bsz)
    aux_t = auxiliary_features.T
    row = lambda v: v.reshape(1, -1)

    grid = (Bsz // _C,)
    full = lambda a: pl.BlockSpec(a.shape, lambda i: (0,) * a.ndim)

    emb_t, wts_t, mask_t = pl.pallas_call(
        _kmote_block,
        grid=grid,
        in_specs=[
            pl.BlockSpec((1, _C), lambda i: (0, i)),
            pl.BlockSpec((_D, _C), lambda i: (0, i)),
            full(W1), full(row(b1)), full(W2.T), full(row(b2)),
            full(A_f), full(B_f), full(Wb_s), full(W_s),
            full(row(C_g)), full(row(Sig_g)), full(row(S_w)), full(row(C_w)),
            full(W_g), full(W_w),
        ],
        out_specs=[
            pl.BlockSpec((_E * _D, _C), lambda i: (0, i)),
            pl.BlockSpec((_E, _C), lambda i: (0, i)),
            pl.BlockSpec((_E, _C), lambda i: (0, i)),
        ],
        out_shape=[
            jax.ShapeDtypeStruct((_E * _D, Bsz), jnp.float32),
            jax.ShapeDtypeStruct((_E, Bsz), jnp.float32),
            jax.ShapeDtypeStruct((_E, Bsz), jnp.bool_),
        ],
        scratch_shapes=[pltpu.VMEM((_NFP, _E * _D), jnp.float32)],
    )(t_row, aux_t, W1, row(b1), W2.T, row(b2),
      A_f, B_f, Wb_s, W_s, row(C_g), row(Sig_g), row(S_w), row(C_w),
      W_g, W_w)

    return (emb_t.T, wts_t.T, mask_t.T)


# speedup vs baseline: 1.1794x; 1.0449x over previous
import jax
import jax.numpy as jnp
import numpy as np
from jax import lax
from jax.experimental import pallas as pl
from jax.experimental.pallas import tpu as pltpu

_E = 4
_D = 16
_GRID = 8
_DEG = 3
_C = 8192

_H = 1.2 / _GRID
_NKNOT = _GRID + 2 * _DEG + 1
_KLO = np.float32(-0.1 - _DEG * _H)
_KHI = np.float32(1.1 + _DEG * _H)
_KSTEP = np.float32((_KHI - _KLO) / (_NKNOT - 1))
_EPS = 1e-8
_NB = _GRID + _DEG
_NFP = 80
_OFF_SPL, _OFF_SILU, _OFF_COS, _OFF_SIN, _OFF_G, _OFF_W = 0, 11, 16, 32, 48, 64

_CPOLY = [np.float32(c) for c in (
    0.9999582, -19.730843, 64.66944, -82.37804, 45.595547)]

_DNT = (((0,), (0,)), ((), ()))
_DNR = (((1,), (0,)), ((), ()))


def _cos2pi(w):
    f = w - jnp.floor(w + 0.5)
    y = f * f
    acc = _CPOLY[4]
    for c in (_CPOLY[3], _CPOLY[2], _CPOLY[1], _CPOLY[0]):
        acc = acc * y + c
    return acc


def _kmote_block(t_ref, aux_ref, w1_ref, b1_ref, w2t_ref, b2_ref,
                 af_ref, bf_ref, wbs_ref, ws_ref, cg_ref, sg_ref,
                 sw_ref, cw_ref, wg_ref, ww_ref,
                 emb_ref, wts_ref, mask_ref, wbig_ref, w1p_ref):
    @pl.when(pl.program_id(0) == 0)
    def _init():
        w1p_ref[0:_D, :] = w1_ref[1:_D + 1, :]
        w1p_ref[_D:_D + 1, :] = w1_ref[0:1, :]
        wbig_ref[:, :] = jnp.zeros((_NFP, _E * _D), jnp.float32)
        wbig_ref[_OFF_SPL:_OFF_SPL + _NB, _D:2 * _D] = ws_ref[:, :]
        wbig_ref[_OFF_SILU:_OFF_SILU + 1, _D:2 * _D] = wbs_ref[:, :]
        wbig_ref[_OFF_COS:_OFF_COS + _D, 0:_D] = af_ref[:, :]
        wbig_ref[_OFF_SIN:_OFF_SIN + _D, 0:_D] = bf_ref[:, :]
        wbig_ref[_OFF_G:_OFF_G + _D, 2 * _D:3 * _D] = wg_ref[:, :]
        wbig_ref[_OFF_W:_OFF_W + _D, 3 * _D:4 * _D] = ww_ref[:, :]

    t = t_ref[:, :]
    aux_t = aux_ref[:, :]

    rin = jnp.concatenate([aux_t, t], axis=0)
    h = lax.dot_general(w1p_ref[:, :], rin, _DNT,
                        preferred_element_type=jnp.float32) + b1_ref[:, :].T
    h = jnp.maximum(h, 0.0)
    logits = lax.dot_general(w2t_ref[:, :], h, _DNR,
                             preferred_element_type=jnp.float32) + b2_ref[:, :].T

    m = jnp.max(logits, axis=0, keepdims=True)
    ex = jnp.exp(logits - m)
    w = ex / jnp.sum(ex, axis=0, keepdims=True)

    iota = jax.lax.broadcasted_iota(jnp.int32, w.shape, 0)
    m1 = jnp.max(w, axis=0, keepdims=True)
    i1 = jnp.min(jnp.where(w == m1, iota, _E), axis=0, keepdims=True)
    w2nd = jnp.where(iota == i1, -1.0, w)
    m2 = jnp.max(w2nd, axis=0, keepdims=True)
    i2 = jnp.min(jnp.where(w2nd == m2, iota, _E), axis=0, keepdims=True)
    sel = (iota == i1) | (iota == i2)
    disp = jnp.where(sel, w, 0.0)
    d_f = disp[0:1, :]
    d_s = disp[1:2, :]
    d_g = disp[2:3, :]
    d_w = disp[3:4, :]

    kfreq8 = jax.lax.broadcasted_iota(jnp.int32, (_D // 2, 1), 0).astype(jnp.float32) + 1.0
    kt8 = kfreq8 * t
    c8 = _cos2pi(kt8)
    s8 = _cos2pi(kt8 - 0.25)
    c_8 = c8[_D // 2 - 1:, :]
    s_8 = s8[_D // 2 - 1:, :]
    chi = c_8 * c8 - s_8 * s8
    shi = s_8 * c8 + c_8 * s8
    cosf = jnp.concatenate([c8, chi], axis=0) * d_f
    sinf = jnp.concatenate([s8, shi], axis=0) * d_f

    dg = (t - cg_ref[:, :].T) / sg_ref[:, :].T
    phi = jnp.exp(-0.5 * dg * dg) * d_g

    u = (t - cw_ref[:, :].T) / sw_ref[:, :].T
    psi = _cos2pi(u * np.float32(5.0 / (2.0 * np.pi))) * jnp.exp(-0.5 * u * u) * d_w

    kvec = _KLO + _KSTEP * jax.lax.broadcasted_iota(jnp.int32, (_NB, 1), 0).astype(jnp.float32)
    wspl = (t - kvec) * np.float32(1.0 / _H)
    v = jnp.abs(wspl - 2.0)
    v2 = v * v
    edge = 2.0 - v
    inner = 4.0 + v2 * (3.0 * v - 6.0)
    b3 = jnp.where(v < 1.0, inner, jnp.where(v < 2.0, edge * edge * edge, 0.0))
    basis = b3 * (d_s * np.float32(1.0 / 6.0))

    silu_t = t * (1.0 / (1.0 + jnp.exp(-t))) * d_s

    pad = jnp.zeros((_OFF_COS - _OFF_SILU - 1, t.shape[1]), dtype=jnp.float32)
    feats = jnp.concatenate([basis, silu_t, pad, cosf, sinf, phi, psi], axis=0)

    emb_ref[:, :] = lax.dot_general(wbig_ref[:, :], feats, _DNT,
                                    preferred_element_type=jnp.float32)
    wts_ref[:, :] = w
    mask_ref[:, :] = sel


def kernel(timestamp_input, auxiliary_features, W1, b1, W2, b2, A_f, B_f,
           Wb_s, W_s, C_g, Sig_g, W_g, S_w, C_w, W_w):
    Bsz = timestamp_input.shape[0]
    t_row = timestamp_input.reshape(1, Bsz)
    aux_t = auxiliary_features.T
    row = lambda v: v.reshape(1, -1)

    grid = (Bsz // _C,)
    full = lambda a: pl.BlockSpec(a.shape, lambda i: (0,) * a.ndim)

    emb_t, wts_t, mask_t = pl.pallas_call(
        _kmote_block,
        grid=grid,
        in_specs=[
            pl.BlockSpec((1, _C), lambda i: (0, i)),
            pl.BlockSpec((_D, _C), lambda i: (0, i)),
            full(W1), full(row(b1)), full(W2.T), full(row(b2)),
            full(A_f), full(B_f), full(Wb_s), full(W_s),
            full(row(C_g)), full(row(Sig_g)), full(row(S_w)), full(row(C_w)),
            full(W_g), full(W_w),
        ],
        out_specs=[
            pl.BlockSpec((_E * _D, _C), lambda i: (0, i)),
            pl.BlockSpec((_E, _C), lambda i: (0, i)),
            pl.BlockSpec((_E, _C), lambda i: (0, i)),
        ],
        out_shape=[
            jax.ShapeDtypeStruct((_E * _D, Bsz), jnp.float32),
            jax.ShapeDtypeStruct((_E, Bsz), jnp.float32),
            jax.ShapeDtypeStruct((_E, Bsz), jnp.bool_),
        ],
        scratch_shapes=[pltpu.VMEM((_NFP, _E * _D), jnp.float32),
                        pltpu.VMEM((1 + 16, 32), jnp.float32)],
    )(t_row, aux_t, W1, row(b1), W2.T, row(b2),
      A_f, B_f, Wb_s, W_s, row(C_g), row(Sig_g), row(S_w), row(C_w),
      W_g, W_w)

    return (emb_t.T, wts_t.T, mask_t.T)
